# baseline (speedup 1.0000x reference)
import jax
import jax.numpy as jnp
from jax.experimental import pallas as pl
from jax.experimental.pallas import tpu as pltpu

_N_ATOMS = 10000
_N_PAIRS = 160000
_MAX_NB = 4
_N_ATYPE = 10
_RC = 5.0
_ACSF_NMU = 20
_APSF_NMU = 10
_ACSF_ETA = 100.0
_APSF_ETA = 25.0
_DENSE = 64
_FEAT = _ACSF_NMU * _N_ATYPE + _APSF_NMU * _N_ATYPE + 22

_CHARGES = jnp.array([0.0, 1.0, 3.0, 5.0, 6.0, 7.0, 8.0, 9.0, 11.0, 15.0, 16.0], dtype=jnp.float32)
_CHARGE_IDX = jnp.array([100000, 0, 1, 2, 3, 4, 5, 6, 7, 8, 9], dtype=jnp.int32)
_ZINDEX = jnp.array([1.0, 3.0, 5.0, 6.0, 7.0, 8.0, 9.0, 11.0, 15.0, 16.0], dtype=jnp.float32)

_B = 1000


def _pbc_shift(dr, box, box_inv):
    ds = jnp.dot(dr, box_inv)
    return dr - jnp.dot(jnp.floor(ds + 0.5), box)


def _ln(x, scale, bias, eps=1e-6):
    mu = jnp.mean(x, axis=-1, keepdims=True)
    var = jnp.mean(jnp.square(x - mu), axis=-1, keepdims=True)
    return (x - mu) / jnp.sqrt(var + eps) * scale + bias


def _mlp_body(x_ref, bs_ref, w1, b1, s1, t1, w2, b2, s2, t2, w3, b3, s3, t3, w4, b4, out_ref):
    i = pl.program_id(0)

    @pl.when(i == 0)
    def _():
        out_ref[...] = jnp.zeros((1, 1), jnp.float32)

    x = x_ref[...]
    h = jnp.dot(x, w1[...], preferred_element_type=jnp.float32) + b1[...]
    h = jax.nn.relu(_ln(h, s1[...], t1[...]))
    h = jnp.dot(h, w2[...], preferred_element_type=jnp.float32) + b2[...]
    h = jax.nn.relu(_ln(h, s2[...], t2[...]))
    h = jnp.dot(h, w3[...], preferred_element_type=jnp.float32) + b3[...]
    h = jax.nn.relu(_ln(h, s3[...], t3[...]))
    o = jnp.dot(h, w4[...], preferred_element_type=jnp.float32) + b4[...]
    out_ref[...] += jnp.sum(o * bs_ref[...], axis=(0, 1), keepdims=True)


def _fused_mlp(combined, bscale, W1, b1, s1, t1, W2, b2, s2, t2, W3, b3, s3, t3, W4, b4):
    n = combined.shape[0]
    grid = (n // _B,)
    full = lambda shape: pl.BlockSpec(shape, lambda i: tuple(0 for _ in shape))
    out = pl.pallas_call(
        _mlp_body,
        grid=grid,
        in_specs=[
            pl.BlockSpec((_B, _FEAT), lambda i: (i, 0)),
            pl.BlockSpec((_B, 1), lambda i: (i, 0)),
            full((_FEAT, _DENSE)), full((1, _DENSE)), full((1, _DENSE)), full((1, _DENSE)),
            full((_DENSE, _DENSE)), full((1, _DENSE)), full((1, _DENSE)), full((1, _DENSE)),
            full((_DENSE, _DENSE)), full((1, _DENSE)), full((1, _DENSE)), full((1, _DENSE)),
            full((_DENSE, 1)), full((1, 1)),
        ],
        out_specs=pl.BlockSpec((1, 1), lambda i: (0, 0)),
        out_shape=jax.ShapeDtypeStruct((1, 1), jnp.float32),
    )(combined, bscale,
      W1, b1.reshape(1, -1), s1.reshape(1, -1), t1.reshape(1, -1),
      W2, b2.reshape(1, -1), s2.reshape(1, -1), t2.reshape(1, -1),
      W3, b3.reshape(1, -1), s3.reshape(1, -1), t3.reshape(1, -1),
      W4, b4.reshape(1, 1))
    return out[0, 0]


def kernel(pos, box, valid_mask, W1, b1, ln1_s, ln1_b, W2, b2, ln2_s, ln2_b, W3, b3, ln3_s, ln3_b, W4, b4, pairs, topo_nblist, topo_mask, mol_ID, atype_indices):
    acsf_mus = jnp.linspace(0.0, 5.0, _ACSF_NMU)
    apsf_mus = jnp.linspace(-1.0, 1.0, _APSF_NMU)
    p = pairs[:, :2]
    dp = jnp.where(p[:, 1] - p[:, 0] <= 0, 1, 0)
    p2 = p - jnp.stack((dp, 2 * dp), axis=1)
    buffer_scales = jnp.where(p2[:, 0] < p2[:, 1], 1.0, 0.0).astype(jnp.float32) * valid_mask
    box_inv = jnp.linalg.inv(box)
    ri = pos[p2[:, 0]]
    rj = pos[p2[:, 1]]
    rij = _pbc_shift(rj - ri, box, box_inv)
    dr_norm = jnp.linalg.norm(rij + 1e-10, axis=1)
    same_mol = mol_ID[p2[:, 0]] == mol_ID[p2[:, 1]]
    buffer_inter = jnp.where(same_mol, 0.0, 1.0)
    cut = 0.5 * (1.0 + jnp.cos(jnp.pi * dr_norm / _RC))
    cut = jnp.where(dr_norm <= _RC, cut, 0.0)
    bscale = buffer_inter * buffer_scales * cut
    jc = p2[:, 0]
    kc = p2[:, 1]
    j_list = topo_nblist[jc]
    k_list = topo_nblist[kc]
    mask_j = (j_list != jc[:, None]) & (j_list != kc[:, None]) & (j_list != -1)
    mask_k = (k_list != jc[:, None]) & (k_list != kc[:, None]) & (k_list != -1)
    j_mask = (topo_mask[jc] & mask_j).astype(jnp.float32)
    k_mask = (topo_mask[kc] & mask_k).astype(jnp.float32)
    rj_env = jnp.where(j_mask[..., None] > 0, pos[j_list], 0.0)
    rj_X = _pbc_shift(rj_env - ri[:, None, :], box, box_inv)
    rj_X_norm = rj_X / jnp.linalg.norm(rj_X + 1e-10, axis=2, keepdims=True)
    rij_unit = rij / (dr_norm[:, None] + 1e-10)
    cos_gamma_i = jnp.einsum('aji,ai->aj', rj_X_norm, rij_unit) * j_mask
    rk_env = jnp.where(k_mask[..., None] > 0, pos[k_list], 0.0)
    rk_X = _pbc_shift(rk_env - rj[:, None, :], box, box_inv)
    rk_X_norm = rk_X / jnp.linalg.norm(rk_X + 1e-10, axis=2, keepdims=True)
    cos_gamma_j = jnp.einsum('aji,ai->aj', rk_X_norm, -rij_unit) * k_mask
    ang_i = jnp.exp(-_APSF_ETA * jnp.square(cos_gamma_i[..., None] - apsf_mus))
    ang_j = jnp.exp(-_APSF_ETA * jnp.square(cos_gamma_j[..., None] - apsf_mus))
    oh_i = (atype_indices[j_list][..., None] == jnp.arange(_N_ATYPE)).astype(jnp.float32)
    oh_j = (atype_indices[k_list][..., None] == jnp.arange(_N_ATYPE)).astype(jnp.float32)
    G_i = jnp.einsum('ijk,ijl->ikl', ang_i * j_mask[..., None], oh_i)
    G_j = jnp.einsum('ijk,ijl->ikl', ang_j * k_mask[..., None], oh_j)
    pair_feat = (G_i + G_j) * 0.5 * bscale[:, None, None]
    r_env = pos[topo_nblist]
    dr = _pbc_shift(r_env - pos[:, None, :], box, box_inv)
    d = jnp.linalg.norm(dr + 1e-10, axis=2)
    x = d / _RC
    f_cut = jnp.where(x < 1.0, 0.5 * (jnp.cos(jnp.pi * x) + 1.0), 0.0) * topo_mask.astype(jnp.float32)
    exp_term = jnp.exp(-_ACSF_ETA * jnp.square(d[..., None] - acsf_mus))
    G_raw = exp_term * f_cut[..., None]
    oh_c = (atype_indices[topo_nblist][..., None] == jnp.arange(_N_ATYPE)).astype(jnp.float32)
    G_atom = jnp.einsum('ijk,ijl->ikl', G_raw, oh_c)
    atom_feat = (G_atom[p2[:, 0]] + G_atom[p2[:, 1]]) * 0.5
    elem = _ZINDEX[atype_indices]
    j_at = elem[p2[:, 0]]
    k_at = elem[p2[:, 1]]
    zi_j = jnp.take(_CHARGE_IDX, jnp.searchsorted(_CHARGES, j_at))
    zi_k = jnp.take(_CHARGE_IDX, jnp.searchsorted(_CHARGES, k_at))
    j_oh = jnp.concatenate([j_at.reshape(-1, 1), jax.nn.one_hot(zi_j, 10, dtype=jnp.float32)], axis=1)
    k_oh = jnp.concatenate([k_at.reshape(-1, 1), jax.nn.one_hot(zi_k, 10, dtype=jnp.float32)], axis=1)
    atype_oh = jnp.concatenate([j_oh, k_oh], axis=1)
    combined = jnp.concatenate([atom_feat.reshape(atom_feat.shape[0], -1), pair_feat.reshape(pair_feat.shape[0], -1), atype_oh], axis=1)
    return _fused_mlp(combined, bscale.reshape(-1, 1),
                      W1, b1, ln1_s, ln1_b, W2, b2, ln2_s, ln2_b,
                      W3, b3, ln3_s, ln3_b, W4, b4)
